# Initial kernel scaffold; baseline (speedup 1.0000x reference)
import jax
import jax.numpy as jnp
from jax.experimental import pallas as pl


def kernel(X, C):
    raise NotImplementedError("write your pallas kernel here")


# speedup vs baseline: 6.5214x; 6.5214x over previous
import functools

import jax
import jax.numpy as jnp
import numpy as np
from jax.experimental import pallas as pl
from jax.experimental.pallas import tpu as pltpu

NUM_NEIGHBORS = 32
MAX_FLOAT = float(np.finfo(np.float32).max)
N = 4096
R = 256


def _body(xt_ref, xn_ref, cc_ref, cr_ref, idx_ref, msk_ref):
    xt = xt_ref[0]
    xn = xn_ref[0]
    xc = []
    xr = []
    for d in range(3):
        xc.append(((xt[d] + xt[3 + d]) + xt[6 + d] + xt[9 + d]) * 0.25)
        xr.append(((xn[:, d:d + 1] + xn[:, 3 + d:4 + d]) + xn[:, 6 + d:7 + d]
                   + xn[:, 9 + d:10 + d]) * 0.25)
    sq_j = (xc[0] * xc[0] + xc[1] * xc[1] + xc[2] * xc[2])[None, :]
    sq_i = xr[0] * xr[0] + xr[1] * xr[1] + xr[2] * xr[2]
    a = jnp.concatenate(
        [xr[0], xr[1], xr[2]] + [jnp.zeros_like(xr[0])] * 5, axis=1
    ).astype(jnp.bfloat16)
    b = jnp.concatenate(
        [xc[0][None, :], xc[1][None, :], xc[2][None, :]]
        + [jnp.zeros_like(xc[0][None, :])] * 5, axis=0
    ).astype(jnp.bfloat16)
    inner = jax.lax.dot_general(
        a, b, (((1,), (0,)), ((), ())),
        preferred_element_type=jnp.float32)
    d2 = jnp.maximum(sq_i + sq_j - 2.0 * inner, 0.0)

    valid = (cc_ref[0] > 0) & (cr_ref[0] > 0)
    vals = jnp.where(valid, d2, MAX_FLOAT)

    iota = jax.lax.broadcasted_iota(jnp.int32, (R, N), 1)
    idx_cols = []
    msk_cols = []
    for _ in range(NUM_NEIGHBORS):
        mn = jnp.min(vals, axis=1, keepdims=True)
        am = jnp.min(jnp.where(vals == mn, iota, N), axis=1, keepdims=True)
        idx_cols.append(am)
        msk_cols.append((mn < MAX_FLOAT).astype(jnp.float32))
        vals = jnp.where(iota == am, jnp.inf, vals)
    idx_ref[0] = jnp.concatenate(idx_cols, axis=1)
    msk_ref[0] = jnp.concatenate(msk_cols, axis=1)


@jax.jit
def kernel(X, C):
    B = X.shape[0]
    Xf = X.reshape(B, N, 12)
    Xt = Xf.transpose(0, 2, 1)
    Ci = (C > 0).astype(jnp.int32)
    Cc = Ci.reshape(B, 1, N)
    Cr = Ci.reshape(B, N, 1)

    grid = (B, N // R)
    out = pl.pallas_call(
        _body,
        grid=grid,
        in_specs=[
            pl.BlockSpec((1, 12, N), lambda b, r: (b, 0, 0)),
            pl.BlockSpec((1, R, 12), lambda b, r: (b, r, 0)),
            pl.BlockSpec((1, 1, N), lambda b, r: (b, 0, 0)),
            pl.BlockSpec((1, R, 1), lambda b, r: (b, r, 0)),
        ],
        out_specs=[
            pl.BlockSpec((1, R, NUM_NEIGHBORS), lambda b, r: (b, r, 0)),
            pl.BlockSpec((1, R, NUM_NEIGHBORS), lambda b, r: (b, r, 0)),
        ],
        out_shape=[
            jax.ShapeDtypeStruct((B, N, NUM_NEIGHBORS), jnp.int32),
            jax.ShapeDtypeStruct((B, N, NUM_NEIGHBORS), jnp.float32),
        ],
        compiler_params=pltpu.CompilerParams(
            dimension_semantics=("arbitrary", "arbitrary"),
        ),
    )(Xt, Xf, Cc, Cr)
    return out[0], out[1]
